# baseline (speedup 1.0000x reference)
import jax
import jax.numpy as jnp
from jax.experimental import pallas as pl
from jax.experimental.pallas import tpu as pltpu

_N = 16384
_R = 128
_C = 128
_MEAN = (_N - 1) / 2.0
_SUMSQ = float(_N) * (float(_N) ** 2 - 1.0) / 12.0


def _xor_sub(x, j):
    nb = _R // (2 * j)
    x4 = x.reshape(nb, 2, j, _C)
    return jnp.concatenate([x4[:, 1:2], x4[:, 0:1]], axis=1).reshape(_R, _C)


def _xor_lane(x, j, idx):
    return jnp.take_along_axis(x, idx, axis=1)


def _stage_split(key, pay, j, sel_a):
    nb = _R // (2 * j)
    k4 = key.reshape(nb, 2, j, _C)
    p4 = pay.reshape(nb, 2, j, _C)
    ka, kb = k4[:, 0], k4[:, 1]
    pa, pb = p4[:, 0], p4[:, 1]
    mn = jnp.minimum(ka, kb)
    mx = jnp.maximum(ka, kb)
    if sel_a is None:
        na, nbv = mn, mx
    else:
        na = jnp.where(sel_a, mn, mx)
        nbv = jnp.where(sel_a, mx, mn)
    pa2 = jnp.where(na != ka, pb, pa)
    pb2 = jnp.where(nbv != kb, pa, pb)
    key = jnp.stack([na, nbv], axis=1).reshape(_R, _C)
    pay = jnp.stack([pa2, pb2], axis=1).reshape(_R, _C)
    return key, pay


def _bitpat(b, rb, cb):
    if b <= 3:
        return "r", rb[b + 3]
    if b <= 6:
        return "c", cb[b - 4]
    if b <= 9:
        return "r", rb[b - 7]
    return "c", cb[b - 7]


def _bitonic_sort(key, pay, rb, cb):
    for lk in range(1, 15):
        for lj in range(lk - 1, -1, -1):
            if lj <= 3:
                jr = 8 << lj
                nb = _R // (2 * jr)
                if lk == 14:
                    sel_a = None
                else:
                    kind, pat = _bitpat(lk, rb, cb)
                    if kind == "r":
                        sel_a = pat.reshape(nb, 2, jr, 1)[:, 0] == 0
                    else:
                        sel_a = (pat == 0).reshape(1, 1, _C)
                key, pay = _stage_split(key, pay, jr, sel_a)
                continue
            _, bj = _bitpat(lj, rb, cb)
            if lk == 14:
                sel = bj == 0
            else:
                _, bk = _bitpat(lk, rb, cb)
                sel = bj == bk
            if 7 <= lj <= 9:
                jr = 1 << (lj - 7)
                p_key = _xor_sub(key, jr)
                p_pay = _xor_sub(pay, jr)
            else:
                s = (1 << (lj - 4)) if lj <= 6 else (1 << (lj - 7))
                idx = jnp.broadcast_to(
                    jax.lax.broadcasted_iota(jnp.int32, (1, _C), 1) ^ s,
                    (_R, _C))
                p_key = _xor_lane(key, s, idx)
                p_pay = _xor_lane(pay, s, idx)
            new_key = jnp.where(sel, jnp.minimum(key, p_key),
                                jnp.maximum(key, p_key))
            swap = new_key != key
            pay = jnp.where(swap, p_pay, pay)
            key = new_key
    return key, pay


def _body(yp_ref, yt_ref, out_ref):
    rows = jax.lax.broadcasted_iota(jnp.int32, (_R, 1), 0)
    cols = jax.lax.broadcasted_iota(jnp.int32, (1, _C), 1)
    rb = [(rows >> b) & 1 for b in range(7)]
    cb = [(cols >> b) & 1 for b in range(7)]
    flat = (((rows >> 3) & 15) | ((cols & 7) << 4) | ((rows & 7) << 7)
            | ((cols >> 3) << 10)).astype(jnp.float32)

    _, z = _bitonic_sort(yp_ref[...], yt_ref[...], rb, cb)
    _, u = _bitonic_sort(z, jnp.broadcast_to(flat, (_R, _C)), rb, cb)

    num = jnp.sum((flat - _MEAN) * (u - _MEAN))
    loss = 1.0 - num / (jnp.float32(_SUMSQ) + 1e-8)
    out_ref[0, 0] = loss


def kernel(y_pred, y_true):
    yp = y_pred.reshape(_R, _C)
    yt = y_true.reshape(_R, _C)
    out = pl.pallas_call(
        _body,
        in_specs=[
            pl.BlockSpec((_R, _C), lambda: (0, 0)),
            pl.BlockSpec((_R, _C), lambda: (0, 0)),
        ],
        out_specs=pl.BlockSpec(memory_space=pltpu.SMEM),
        out_shape=jax.ShapeDtypeStruct((1, 1), jnp.float32),
        grid=(),
    )(yp, yt)
    return out.reshape(())
